# Initial kernel scaffold; baseline (speedup 1.0000x reference)
import jax
import jax.numpy as jnp
from jax.experimental import pallas as pl


def kernel(x, edge_index, batch, W_enc, b_enc, Wself1, Wneigh1, b1, Wself2, Wneigh2, b2, centers):
    raise NotImplementedError("write your pallas kernel here")


# speedup vs baseline: 2.5777x; 2.5777x over previous
import functools

import jax
import jax.numpy as jnp
from jax import lax
from jax.experimental import pallas as pl
from jax.experimental.pallas import tpu as pltpu
from jax.experimental.pallas import tpu_sc as plsc

N = 10000
D = 128
NE = 4
NG = 16

NC = 2
NS = 16
TILES = NC * NS
CH = 128
ROWS_PER_TILE = 640
RCH = ROWS_PER_TILE // CH
N_PAD = ROWS_PER_TILE * NS

RB = 1000
GRID = N // RB
NB_PAD = 10240


def _sc_mesh():
    return plsc.VectorSubcoreMesh(core_axis_name="c", subcore_axis_name="s",
                                  num_cores=NC, num_subcores=NS)


def _make_agg_h(chunks_per_tile):
    @functools.partial(
        pl.kernel,
        out_type=[
            jax.ShapeDtypeStruct((NC * N_PAD, D), jnp.float32),
            jax.ShapeDtypeStruct((NC * N_PAD, D), jnp.float32),
        ],
        mesh=_sc_mesh(),
        scratch_types=[
            pltpu.VMEM((CH,), jnp.int32),
            pltpu.VMEM((CH,), jnp.int32),
            pltpu.VMEM((CH, D), jnp.float32),
            pltpu.VMEM((CH, D), jnp.float32),
            pltpu.SemaphoreType.DMA,
            pltpu.VMEM_SHARED((N_PAD, D), jnp.float32),
        ],
    )
    def agg(v_hbm, src_hbm, dst_hbm, zrow_hbm, ones_hbm,
            out_hbm, deg_hbm,
            src_idx, dst_idx, rows, ones_v, sem, acc_sh):
        c = lax.axis_index("c")
        s = lax.axis_index("s")
        w = s * NC + c
        r0 = s * ROWS_PER_TILE
        o0 = c * N_PAD + r0
        pltpu.sync_copy(zrow_hbm, rows)
        for j in range(RCH):
            pltpu.sync_copy(rows, acc_sh.at[pl.ds(r0 + j * CH, CH)])
        plsc.subcore_barrier()

        def chunk(k, carry):
            base = (w * chunks_per_tile + k) * CH
            pltpu.sync_copy(src_hbm.at[pl.ds(base, CH)], src_idx)
            pltpu.sync_copy(dst_hbm.at[pl.ds(base, CH)], dst_idx)
            pltpu.async_copy(v_hbm.at[src_idx], rows, sem).wait()
            pltpu.sync_copy(rows, acc_sh.at[dst_idx], add=True)
            return carry

        lax.fori_loop(0, chunks_per_tile, chunk, 0)
        plsc.subcore_barrier()
        for j in range(RCH):
            pltpu.sync_copy(acc_sh.at[pl.ds(r0 + j * CH, CH)], rows)
            pltpu.sync_copy(rows, out_hbm.at[pl.ds(o0 + j * CH, CH)])

        pltpu.sync_copy(zrow_hbm, rows)
        for j in range(RCH):
            pltpu.sync_copy(rows, acc_sh.at[pl.ds(r0 + j * CH, CH)])
        pltpu.sync_copy(ones_hbm, ones_v)
        plsc.subcore_barrier()

        def dchunk(k, carry):
            base = (w * chunks_per_tile + k) * CH
            pltpu.sync_copy(dst_hbm.at[pl.ds(base, CH)], dst_idx)
            pltpu.sync_copy(ones_v, acc_sh.at[dst_idx], add=True)
            return carry

        lax.fori_loop(0, chunks_per_tile, dchunk, 0)
        plsc.subcore_barrier()
        for j in range(RCH):
            pltpu.sync_copy(acc_sh.at[pl.ds(r0 + j * CH, CH)], rows)
            pltpu.sync_copy(rows, deg_hbm.at[pl.ds(o0 + j * CH, CH)])

    return agg


def _make_agg_experts(chunks_per_tile):
    @functools.partial(
        pl.kernel,
        out_type=[jax.ShapeDtypeStruct((NC * N_PAD, D), jnp.float32)
                  for _ in range(NE)],
        mesh=_sc_mesh(),
        scratch_types=[
            pltpu.VMEM((CH,), jnp.int32),
            pltpu.VMEM((CH,), jnp.int32),
            pltpu.VMEM((CH, D), jnp.float32),
            pltpu.SemaphoreType.DMA,
            pltpu.VMEM_SHARED((N_PAD, D), jnp.float32),
        ],
    )
    def agg(v0_hbm, v1_hbm, v2_hbm, v3_hbm, src_hbm, dst_hbm, zrow_hbm,
            o0_hbm, o1_hbm, o2_hbm, o3_hbm,
            src_idx, dst_idx, rows, sem, acc_sh):
        c = lax.axis_index("c")
        s = lax.axis_index("s")
        w = s * NC + c
        r0 = s * ROWS_PER_TILE
        o0 = c * N_PAD + r0
        vs = [v0_hbm, v1_hbm, v2_hbm, v3_hbm]
        os_ = [o0_hbm, o1_hbm, o2_hbm, o3_hbm]
        for e in range(NE):
            pltpu.sync_copy(zrow_hbm, rows)
            for j in range(RCH):
                pltpu.sync_copy(rows, acc_sh.at[pl.ds(r0 + j * CH, CH)])
            plsc.subcore_barrier()

            def chunk(k, carry):
                base = (w * chunks_per_tile + k) * CH
                pltpu.sync_copy(src_hbm.at[pl.ds(base, CH)], src_idx)
                pltpu.sync_copy(dst_hbm.at[pl.ds(base, CH)], dst_idx)
                pltpu.async_copy(vs[e].at[src_idx], rows, sem).wait()
                pltpu.sync_copy(rows, acc_sh.at[dst_idx], add=True)
                return carry

            lax.fori_loop(0, chunks_per_tile, chunk, 0)
            plsc.subcore_barrier()
            for j in range(RCH):
                pltpu.sync_copy(acc_sh.at[pl.ds(r0 + j * CH, CH)], rows)
                pltpu.sync_copy(rows, os_[e].at[pl.ds(o0 + j * CH, CH)])

    return agg


def _encoder_body(x_ref, w_ref, b_ref, bfull_ref, bblk_ref, cent_ref,
                  h_ref, p_ref):
    h = jnp.dot(x_ref[...], w_ref[...], preferred_element_type=jnp.float32)
    h_ref[...] = jnp.maximum(h + b_ref[...], 0.0)
    bf = bfull_ref[...]
    bb = bblk_ref[...]
    inv_logn = 1.0 / jnp.log(jnp.float32(N))
    logn = jnp.zeros((RB, 1), jnp.float32)
    for g in range(NG):
        cnt = jnp.sum(jnp.where(bf == g, 1.0, 0.0))
        lg = jnp.log(jnp.maximum(cnt, 1.0)) * inv_logn
        logn = logn + jnp.where(bb == g, lg, 0.0)
    dlt = logn - cent_ref[...]
    sc = -(dlt * dlt)
    m = jnp.max(sc, axis=1, keepdims=True)
    ex = jnp.exp(sc - m)
    p_ref[...] = ex / jnp.sum(ex, axis=1, keepdims=True)


def _layer1_body(h_ref, acc_ref, deg_ref, ws_ref, wn_ref, b_ref,
                 o0_ref, o1_ref, o2_ref, o3_ref):
    dg = deg_ref[0, :, 0:1] + deg_ref[1, :, 0:1]
    inv = 1.0 / jnp.maximum(dg, 1.0)
    m1 = (acc_ref[0] + acc_ref[1]) * inv
    h = h_ref[...]
    outs = [o0_ref, o1_ref, o2_ref, o3_ref]
    for e in range(NE):
        ye = (jnp.dot(h, ws_ref[e], preferred_element_type=jnp.float32)
              + jnp.dot(m1, wn_ref[e], preferred_element_type=jnp.float32)
              + b_ref[e:e + 1, :])
        outs[e][...] = jnp.maximum(ye, 0.0)


def _layer2_body(h0_ref, h1_ref, h2_ref, h3_ref, a0_ref, a1_ref, a2_ref,
                 a3_ref, deg_ref, p_ref, ws_ref, wn_ref, b_ref, out_ref):
    dg = deg_ref[0, :, 0:1] + deg_ref[1, :, 0:1]
    inv = 1.0 / jnp.maximum(dg, 1.0)
    p = p_ref[...]
    out = jnp.zeros((RB, D), jnp.float32)
    hes = [h0_ref, h1_ref, h2_ref, h3_ref]
    accs = [a0_ref, a1_ref, a2_ref, a3_ref]
    for e in range(NE):
        m2 = (accs[e][0] + accs[e][1]) * inv
        ye = (jnp.dot(hes[e][...], ws_ref[e], preferred_element_type=jnp.float32)
              + jnp.dot(m2, wn_ref[e], preferred_element_type=jnp.float32)
              + b_ref[e:e + 1, :])
        out = out + p[:, e:e + 1] * ye
    out_ref[...] = out


def kernel(x, edge_index, batch, W_enc, b_enc, Wself1, Wneigh1, b1,
           Wself2, Wneigh2, b2, centers):
    src = edge_index[0].astype(jnp.int32)
    dst = edge_index[1].astype(jnp.int32)
    e_edges = src.shape[0]
    chunks_per_tile = -(-e_edges // (TILES * CH))
    e_pad = chunks_per_tile * TILES * CH
    npad = e_pad - e_edges
    src_p = jnp.concatenate([src, jnp.zeros((npad,), jnp.int32)])
    dst_p = jnp.concatenate(
        [dst, N + (jnp.arange(npad, dtype=jnp.int32) % CH)])
    zrow = jnp.zeros((CH, D), jnp.float32)
    ones128 = jnp.ones((CH, D), jnp.float32)

    batch_i = batch.astype(jnp.int32)
    batch_full = jnp.concatenate(
        [batch_i, jnp.full((NB_PAD - N,), NG, jnp.int32)]).reshape(1, NB_PAD)
    batch_blk = batch_i.reshape(N, 1)

    h, probs = pl.pallas_call(
        _encoder_body,
        grid=(GRID,),
        in_specs=[
            pl.BlockSpec((RB, D), lambda i: (i, 0)),
            pl.BlockSpec((D, D), lambda i: (0, 0)),
            pl.BlockSpec((1, D), lambda i: (0, 0)),
            pl.BlockSpec((1, NB_PAD), lambda i: (0, 0)),
            pl.BlockSpec((RB, 1), lambda i: (i, 0)),
            pl.BlockSpec((1, NE), lambda i: (0, 0)),
        ],
        out_specs=[
            pl.BlockSpec((RB, D), lambda i: (i, 0)),
            pl.BlockSpec((RB, NE), lambda i: (i, 0)),
        ],
        out_shape=[
            jax.ShapeDtypeStruct((N, D), jnp.float32),
            jax.ShapeDtypeStruct((N, NE), jnp.float32),
        ],
    )(x, W_enc, b_enc.reshape(1, D), batch_full, batch_blk,
      centers.reshape(1, NE))

    acc1_f, deg_f = _make_agg_h(chunks_per_tile)(
        h, src_p, dst_p, zrow, ones128)
    acc1 = acc1_f.reshape(NC, N_PAD, D)
    deg = deg_f.reshape(NC, N_PAD, D)

    hes = pl.pallas_call(
        _layer1_body,
        grid=(GRID,),
        in_specs=[
            pl.BlockSpec((RB, D), lambda i: (i, 0)),
            pl.BlockSpec((NC, RB, D), lambda i: (0, i, 0)),
            pl.BlockSpec((NC, RB, D), lambda i: (0, i, 0)),
            pl.BlockSpec((NE, D, D), lambda i: (0, 0, 0)),
            pl.BlockSpec((NE, D, D), lambda i: (0, 0, 0)),
            pl.BlockSpec((NE, D), lambda i: (0, 0)),
        ],
        out_specs=[pl.BlockSpec((RB, D), lambda i: (i, 0))
                   for _ in range(NE)],
        out_shape=[jax.ShapeDtypeStruct((N, D), jnp.float32)
                   for _ in range(NE)],
    )(h, acc1, deg, Wself1, Wneigh1, b1)

    acc2_fs = _make_agg_experts(chunks_per_tile)(
        hes[0], hes[1], hes[2], hes[3], src_p, dst_p, zrow)
    acc2s = [a.reshape(NC, N_PAD, D) for a in acc2_fs]

    out = pl.pallas_call(
        _layer2_body,
        grid=(GRID,),
        in_specs=(
            [pl.BlockSpec((RB, D), lambda i: (i, 0)) for _ in range(NE)]
            + [pl.BlockSpec((NC, RB, D), lambda i: (0, i, 0))
               for _ in range(NE)]
            + [
                pl.BlockSpec((NC, RB, D), lambda i: (0, i, 0)),
                pl.BlockSpec((RB, NE), lambda i: (i, 0)),
                pl.BlockSpec((NE, D, D), lambda i: (0, 0, 0)),
                pl.BlockSpec((NE, D, D), lambda i: (0, 0, 0)),
                pl.BlockSpec((NE, D), lambda i: (0, 0)),
            ]
        ),
        out_specs=pl.BlockSpec((RB, D), lambda i: (i, 0)),
        out_shape=jax.ShapeDtypeStruct((N, D), jnp.float32),
    )(hes[0], hes[1], hes[2], hes[3], acc2s[0], acc2s[1], acc2s[2],
      acc2s[3], deg, probs, Wself2, Wneigh2, b2)
    return out
